# Initial kernel scaffold; baseline (speedup 1.0000x reference)
import jax
import jax.numpy as jnp
from jax.experimental import pallas as pl


def kernel(x, edge_index, batch, pos, num_graphs, W1, att_src1, att_dst1, b1, W2, att_src2, att_dst2, b2, fc_w, fc_b):
    raise NotImplementedError("write your pallas kernel here")


# speedup vs baseline: 12.7553x; 12.7553x over previous
import functools
import math

import jax
import jax.numpy as jnp
from jax import lax
from jax.experimental import pallas as pl
from jax.experimental.pallas import tpu as pltpu
from jax.experimental.pallas import tpu_sc as plsc

G = 8
NCORES = 2
NSUB = 16
NW = NCORES * NSUB
K = 128


def _gelu(x):
    return x * 0.5 * (1.0 + lax.erf(x / math.sqrt(2.0)))


def _mm1_body(x_ref, w_ref, as_ref, ad_ref,
              h_ref, als_ref, ald_ref, mxs_ref, mxd_ref):
    i = pl.program_id(0)

    @pl.when(i == 0)
    def _init():
        mxs_ref[...] = jnp.full_like(mxs_ref, -1e30)
        mxd_ref[...] = jnp.full_like(mxd_ref, -1e30)

    h = jnp.dot(x_ref[...], w_ref[...], preferred_element_type=jnp.float32)
    h_ref[...] = h
    als = jnp.dot(h, as_ref[...], preferred_element_type=jnp.float32)
    ald = jnp.dot(h, ad_ref[...], preferred_element_type=jnp.float32)
    als_ref[...] = als
    ald_ref[...] = ald
    mxs_ref[...] = jnp.maximum(mxs_ref[...], jnp.max(als))
    mxd_ref[...] = jnp.maximum(mxd_ref[...], jnp.max(ald))


def _mm1(x, W, a_s, a_d, BN):
    N, DIN = x.shape
    DH = W.shape[1]
    return pl.pallas_call(
        _mm1_body,
        grid=(N // BN,),
        in_specs=[
            pl.BlockSpec((BN, DIN), lambda i: (i, 0)),
            pl.BlockSpec((DIN, DH), lambda i: (0, 0)),
            pl.BlockSpec((DH, 1), lambda i: (0, 0)),
            pl.BlockSpec((DH, 1), lambda i: (0, 0)),
        ],
        out_specs=[
            pl.BlockSpec((BN, DH), lambda i: (i, 0)),
            pl.BlockSpec((BN, 1), lambda i: (i, 0)),
            pl.BlockSpec((BN, 1), lambda i: (i, 0)),
            pl.BlockSpec((1, 128), lambda i: (0, 0)),
            pl.BlockSpec((1, 128), lambda i: (0, 0)),
        ],
        out_shape=[
            jax.ShapeDtypeStruct((N, DH), jnp.float32),
            jax.ShapeDtypeStruct((N, 1), jnp.float32),
            jax.ShapeDtypeStruct((N, 1), jnp.float32),
            jax.ShapeDtypeStruct((1, 128), jnp.float32),
            jax.ShapeDtypeStruct((1, 128), jnp.float32),
        ],
    )(x, W, a_s.reshape(DH, 1), a_d.reshape(DH, 1))


def _mm2_body(p_ref, d_ref, b_ref, w_ref, as_ref, ad_ref,
              h2_ref, als_ref, ald_ref, mxs_ref, mxd_ref):
    i = pl.program_id(0)

    @pl.when(i == 0)
    def _init():
        mxs_ref[...] = jnp.full_like(mxs_ref, -1e30)
        mxd_ref[...] = jnp.full_like(mxd_ref, -1e30)

    agg = p_ref[0] + p_ref[1]
    den = d_ref[0] + d_ref[1]
    out1 = agg / jnp.maximum(den, 1e-16) + b_ref[...]
    g = _gelu(out1)
    h2 = jnp.dot(g, w_ref[...], preferred_element_type=jnp.float32)
    h2_ref[...] = h2
    als = jnp.dot(h2, as_ref[...], preferred_element_type=jnp.float32)
    ald = jnp.dot(h2, ad_ref[...], preferred_element_type=jnp.float32)
    als_ref[...] = als
    ald_ref[...] = ald
    mxs_ref[...] = jnp.maximum(mxs_ref[...], jnp.max(als))
    mxd_ref[...] = jnp.maximum(mxd_ref[...], jnp.max(ald))


def _mm2(parts, dens, b, W, a_s, a_d, N, BN):
    DH = W.shape[0]
    return pl.pallas_call(
        _mm2_body,
        grid=(N // BN,),
        in_specs=[
            pl.BlockSpec((2, BN, DH), lambda i: (0, i, 0)),
            pl.BlockSpec((2, BN, 1), lambda i: (0, i, 0)),
            pl.BlockSpec((1, DH), lambda i: (0, 0)),
            pl.BlockSpec((DH, DH), lambda i: (0, 0)),
            pl.BlockSpec((DH, 1), lambda i: (0, 0)),
            pl.BlockSpec((DH, 1), lambda i: (0, 0)),
        ],
        out_specs=[
            pl.BlockSpec((BN, DH), lambda i: (i, 0)),
            pl.BlockSpec((BN, 1), lambda i: (i, 0)),
            pl.BlockSpec((BN, 1), lambda i: (i, 0)),
            pl.BlockSpec((1, 128), lambda i: (0, 0)),
            pl.BlockSpec((1, 128), lambda i: (0, 0)),
        ],
        out_shape=[
            jax.ShapeDtypeStruct((N, DH), jnp.float32),
            jax.ShapeDtypeStruct((N, 1), jnp.float32),
            jax.ShapeDtypeStruct((N, 1), jnp.float32),
            jax.ShapeDtypeStruct((1, 128), jnp.float32),
            jax.ShapeDtypeStruct((1, 128), jnp.float32),
        ],
    )(parts, dens, b.reshape(1, DH), W, a_s.reshape(DH, 1), a_d.reshape(DH, 1))


def _pool_body(nb, p_ref, d_ref, b_ref, bat_ref, pos_ref, fw_ref, fb_ref,
               pooled_ref, score_ref, sums, cnts):
    i = pl.program_id(0)

    @pl.when(i == 0)
    def _init():
        sums[...] = jnp.zeros_like(sums)
        cnts[...] = jnp.zeros_like(cnts)

    logits = (p_ref[0] + p_ref[1]) / jnp.maximum(d_ref[0] + d_ref[1], 1e-16)
    logits = logits + b_ref[...]
    w = pos_ref[0]
    bat = bat_ref[0]
    gids = lax.broadcasted_iota(jnp.int32, (G, 1), 0)
    mask = jnp.where(bat == gids, 1.0, 0.0) * w
    sums[...] += jnp.dot(mask, logits, preferred_element_type=jnp.float32)
    cnts[...] += jnp.broadcast_to(
        jnp.sum(mask, axis=1, keepdims=True), cnts.shape)

    @pl.when(i == nb - 1)
    def _fin():
        pooled = sums[...] / jnp.maximum(cnts[...], 1.0)
        pooled_ref[...] = pooled
        gp = _gelu(pooled)
        score_ref[...] = (
            jnp.dot(gp, fw_ref[...], preferred_element_type=jnp.float32)
            + fb_ref[...])


def _pool(parts, dens, b, batch3, pos3, fc_w, fc_b, N, BN):
    DH = b.shape[0]
    nb = N // BN
    return pl.pallas_call(
        functools.partial(_pool_body, nb),
        grid=(nb,),
        in_specs=[
            pl.BlockSpec((2, BN, DH), lambda i: (0, i, 0)),
            pl.BlockSpec((2, BN, 1), lambda i: (0, i, 0)),
            pl.BlockSpec((1, DH), lambda i: (0, 0)),
            pl.BlockSpec((1, 1, BN), lambda i: (i, 0, 0)),
            pl.BlockSpec((1, 1, BN), lambda i: (i, 0, 0)),
            pl.BlockSpec((DH, 1), lambda i: (0, 0)),
            pl.BlockSpec((1, 1), lambda i: (0, 0)),
        ],
        out_specs=[
            pl.BlockSpec((G, DH), lambda i: (0, 0)),
            pl.BlockSpec((G, 1), lambda i: (0, 0)),
        ],
        out_shape=[
            jax.ShapeDtypeStruct((G, DH), jnp.float32),
            jax.ShapeDtypeStruct((G, 1), jnp.float32),
        ],
        scratch_shapes=[
            pltpu.VMEM((G, DH), jnp.float32),
            pltpu.VMEM((G, DH), jnp.float32),
        ],
    )(parts, dens, b.reshape(1, DH), batch3, pos3, fc_w,
      fc_b.reshape(1, 1))


def _make_sc_edge(N, NPD, DH, NB, E_tot):
    TR = NPD // NSUB
    mesh = plsc.VectorSubcoreMesh(core_axis_name="c", subcore_axis_name="s")

    @functools.partial(
        pl.kernel,
        out_type=(
            jax.ShapeDtypeStruct((NCORES * NPD, DH), jnp.float32),
            jax.ShapeDtypeStruct((NCORES * NPD,), jnp.float32),
        ),
        mesh=mesh,
        compiler_params=pltpu.CompilerParams(needs_layout_passes=False),
        scratch_types=[
            pltpu.VMEM((NB, K), jnp.int32),
            pltpu.VMEM((NB, K), jnp.int32),
            pltpu.VMEM((K,), jnp.float32),
            pltpu.VMEM((K,), jnp.float32),
            pltpu.VMEM((K,), jnp.float32),
            pltpu.VMEM((K, DH), jnp.float32),
            pltpu.VMEM((128,), jnp.float32),
            pltpu.VMEM((128,), jnp.float32),
            pltpu.VMEM((TR,), jnp.float32),
            pltpu.VMEM_SHARED((NPD, DH), jnp.float32),
            pltpu.VMEM_SHARED((NPD,), jnp.float32),
            pltpu.SemaphoreType.DMA,
        ],
    )
    def edge_kernel(h_hbm, als_hbm, ald_hbm, mxs_hbm, mxd_hbm,
                    srcr, dstr, zrow, zden,
                    accp, denp,
                    src_v, dst_v, asg, adg, exb, rows_v, shs, shd,
                    den_stage, acc_sh, den_sh, sem):
        cid = lax.axis_index("c")
        sid = lax.axis_index("s")
        wid = cid * NSUB + sid

        pltpu.sync_copy(zrow, acc_sh.at[pl.ds(sid * TR, TR)])
        pltpu.sync_copy(zden, den_stage)
        pltpu.sync_copy(den_stage, den_sh.at[pl.ds(sid * TR, TR)])

        pltpu.sync_copy(srcr.at[wid], src_v)
        pltpu.sync_copy(dstr.at[wid], dst_v)
        pltpu.sync_copy(mxs_hbm, shs)
        pltpu.sync_copy(mxd_hbm, shd)
        shift = shs[pl.ds(0, 16)] + shd[pl.ds(0, 16)]
        plsc.subcore_barrier()

        def _batch(j, carry):
            pltpu.async_copy(als_hbm.at[src_v.at[j]], asg, sem).wait()
            pltpu.async_copy(ald_hbm.at[dst_v.at[j]], adg, sem).wait()
            for g in range(K // 16):
                sl = pl.ds(g * 16, 16)
                e = asg[sl] + adg[sl]
                e = jnp.where(e >= 0.0, e, 0.2 * e) - shift
                ex = jnp.exp(e)
                eid = ((wid * NB + j) * K + g * 16
                       + lax.broadcasted_iota(jnp.int32, (16,), 0))
                exb[sl] = jnp.where(eid < E_tot, ex, 0.0)
            pltpu.sync_copy(exb, den_sh.at[dst_v.at[j]], add=True)

            pltpu.async_copy(h_hbm.at[src_v.at[j]], rows_v, sem).wait()

            def _scale(g2, c2):
                exg = exb[pl.ds(g2 * 16, 16)]
                for kk in range(16):
                    sv16 = jnp.full((16,), exg[kk], jnp.float32)
                    row = g2 * 16 + kk
                    for c in range(DH // 16):
                        rows_v[row, pl.ds(c * 16, 16)] = (
                            rows_v[row, pl.ds(c * 16, 16)] * sv16)
                return c2

            lax.fori_loop(0, K // 16, _scale, 0)
            pltpu.sync_copy(rows_v, acc_sh.at[dst_v.at[j]], add=True)
            return carry

        lax.fori_loop(0, NB, _batch, 0)
        plsc.subcore_barrier()

        pltpu.sync_copy(acc_sh.at[pl.ds(sid * TR, TR)],
                        accp.at[pl.ds(cid * NPD + sid * TR, TR)])
        pltpu.sync_copy(den_sh.at[pl.ds(sid * TR, TR)], den_stage)
        pltpu.sync_copy(den_stage,
                        denp.at[pl.ds(cid * NPD + sid * TR, TR)])

    return edge_kernel


def kernel(x, edge_index, batch, pos, num_graphs, W1, att_src1, att_dst1, b1,
           W2, att_src2, att_dst2, b2, fc_w, fc_b):
    N, DIN = x.shape
    DH = W1.shape[1]
    E = edge_index.shape[1]
    E_tot = E + N
    BN = 1000

    NB = -(-E_tot // (NW * K))
    EPAD = NW * NB * K
    TRD = ((-(-N // NSUB)) + 7) // 8 * 8
    NPD = NSUB * TRD

    loop = jnp.arange(N, dtype=jnp.int32)
    pad = jnp.zeros((EPAD - E_tot,), dtype=jnp.int32)
    srcr = jnp.concatenate([edge_index[0], loop, pad]).reshape(NW, NB, K)
    dstr = jnp.concatenate([edge_index[1], loop, pad]).reshape(NW, NB, K)
    zrow = jnp.zeros((NPD // NSUB, DH), jnp.float32)
    zden = jnp.zeros((TRD,), jnp.float32)
    batch3 = batch.astype(jnp.int32).reshape(N // BN, 1, BN)
    pos3 = pos.astype(jnp.float32).reshape(N // BN, 1, BN)

    sc_edge = _make_sc_edge(N, NPD, DH, NB, E_tot)

    h1, als1, ald1, mxs1, mxd1 = _mm1(x, W1, att_src1, att_dst1, BN)
    accp1, denp1 = sc_edge(h1, als1.reshape(N), ald1.reshape(N),
                           mxs1.reshape(128), mxd1.reshape(128),
                           srcr, dstr, zrow, zden)
    parts1 = accp1.reshape(NCORES, NPD, DH)[:, :N]
    dens1 = denp1.reshape(NCORES, NPD, 1)[:, :N]

    h2, als2, ald2, mxs2, mxd2 = _mm2(parts1, dens1, b1, W2,
                                      att_src2, att_dst2, N, BN)
    accp2, denp2 = sc_edge(h2, als2.reshape(N), ald2.reshape(N),
                           mxs2.reshape(128), mxd2.reshape(128),
                           srcr, dstr, zrow, zden)
    parts2 = accp2.reshape(NCORES, NPD, DH)[:, :N]
    dens2 = denp2.reshape(NCORES, NPD, 1)[:, :N]

    pooled, score = _pool(parts2, dens2, b2, batch3, pos3, fc_w, fc_b, N, BN)
    return (score.reshape(1, G), pooled)


# speedup vs baseline: 19.0359x; 1.4924x over previous
import functools
import math

import jax
import jax.numpy as jnp
from jax import lax
from jax.experimental import pallas as pl
from jax.experimental.pallas import tpu as pltpu
from jax.experimental.pallas import tpu_sc as plsc

G = 8
NCORES = 2
NSUB = 16
NW = NCORES * NSUB
K = 128


def _gelu(x):
    return x * 0.5 * (1.0 + lax.erf(x / math.sqrt(2.0)))


def _mm1_body(x_ref, w_ref, as_ref, ad_ref,
              h_ref, als_ref, ald_ref, mxs_ref, mxd_ref):
    i = pl.program_id(0)

    @pl.when(i == 0)
    def _init():
        mxs_ref[...] = jnp.full_like(mxs_ref, -1e30)
        mxd_ref[...] = jnp.full_like(mxd_ref, -1e30)

    h = jnp.dot(x_ref[...], w_ref[...], preferred_element_type=jnp.float32)
    h_ref[...] = h
    als = jnp.dot(h, as_ref[...], preferred_element_type=jnp.float32)
    ald = jnp.dot(h, ad_ref[...], preferred_element_type=jnp.float32)
    als_ref[...] = als
    ald_ref[...] = ald
    mxs_ref[...] = jnp.maximum(mxs_ref[...], jnp.max(als))
    mxd_ref[...] = jnp.maximum(mxd_ref[...], jnp.max(ald))


def _mm1(x, W, a_s, a_d, BN):
    N, DIN = x.shape
    DH = W.shape[1]
    return pl.pallas_call(
        _mm1_body,
        grid=(N // BN,),
        in_specs=[
            pl.BlockSpec((BN, DIN), lambda i: (i, 0)),
            pl.BlockSpec((DIN, DH), lambda i: (0, 0)),
            pl.BlockSpec((DH, 1), lambda i: (0, 0)),
            pl.BlockSpec((DH, 1), lambda i: (0, 0)),
        ],
        out_specs=[
            pl.BlockSpec((BN, DH), lambda i: (i, 0)),
            pl.BlockSpec((BN, 1), lambda i: (i, 0)),
            pl.BlockSpec((BN, 1), lambda i: (i, 0)),
            pl.BlockSpec((1, 128), lambda i: (0, 0)),
            pl.BlockSpec((1, 128), lambda i: (0, 0)),
        ],
        out_shape=[
            jax.ShapeDtypeStruct((N, DH), jnp.float32),
            jax.ShapeDtypeStruct((N, 1), jnp.float32),
            jax.ShapeDtypeStruct((N, 1), jnp.float32),
            jax.ShapeDtypeStruct((1, 128), jnp.float32),
            jax.ShapeDtypeStruct((1, 128), jnp.float32),
        ],
    )(x, W, a_s.reshape(DH, 1), a_d.reshape(DH, 1))


def _mm2_body(p_ref, d_ref, b_ref, w_ref, as_ref, ad_ref,
              h2_ref, als_ref, ald_ref, mxs_ref, mxd_ref):
    i = pl.program_id(0)

    @pl.when(i == 0)
    def _init():
        mxs_ref[...] = jnp.full_like(mxs_ref, -1e30)
        mxd_ref[...] = jnp.full_like(mxd_ref, -1e30)

    agg = p_ref[0] + p_ref[1]
    den = d_ref[0] + d_ref[1]
    out1 = agg / jnp.maximum(den, 1e-16) + b_ref[...]
    g = _gelu(out1)
    h2 = jnp.dot(g, w_ref[...], preferred_element_type=jnp.float32)
    h2_ref[...] = h2
    als = jnp.dot(h2, as_ref[...], preferred_element_type=jnp.float32)
    ald = jnp.dot(h2, ad_ref[...], preferred_element_type=jnp.float32)
    als_ref[...] = als
    ald_ref[...] = ald
    mxs_ref[...] = jnp.maximum(mxs_ref[...], jnp.max(als))
    mxd_ref[...] = jnp.maximum(mxd_ref[...], jnp.max(ald))


def _mm2(parts, dens, b, W, a_s, a_d, N, BN):
    DH = W.shape[0]
    return pl.pallas_call(
        _mm2_body,
        grid=(N // BN,),
        in_specs=[
            pl.BlockSpec((2, BN, DH), lambda i: (0, i, 0)),
            pl.BlockSpec((2, BN, 1), lambda i: (0, i, 0)),
            pl.BlockSpec((1, DH), lambda i: (0, 0)),
            pl.BlockSpec((DH, DH), lambda i: (0, 0)),
            pl.BlockSpec((DH, 1), lambda i: (0, 0)),
            pl.BlockSpec((DH, 1), lambda i: (0, 0)),
        ],
        out_specs=[
            pl.BlockSpec((BN, DH), lambda i: (i, 0)),
            pl.BlockSpec((BN, 1), lambda i: (i, 0)),
            pl.BlockSpec((BN, 1), lambda i: (i, 0)),
            pl.BlockSpec((1, 128), lambda i: (0, 0)),
            pl.BlockSpec((1, 128), lambda i: (0, 0)),
        ],
        out_shape=[
            jax.ShapeDtypeStruct((N, DH), jnp.float32),
            jax.ShapeDtypeStruct((N, 1), jnp.float32),
            jax.ShapeDtypeStruct((N, 1), jnp.float32),
            jax.ShapeDtypeStruct((1, 128), jnp.float32),
            jax.ShapeDtypeStruct((1, 128), jnp.float32),
        ],
    )(parts, dens, b.reshape(1, DH), W, a_s.reshape(DH, 1), a_d.reshape(DH, 1))


def _pool_body(nb, p_ref, d_ref, b_ref, bat_ref, pos_ref, fw_ref, fb_ref,
               pooled_ref, score_ref, sums, cnts):
    i = pl.program_id(0)

    @pl.when(i == 0)
    def _init():
        sums[...] = jnp.zeros_like(sums)
        cnts[...] = jnp.zeros_like(cnts)

    logits = (p_ref[0] + p_ref[1]) / jnp.maximum(d_ref[0] + d_ref[1], 1e-16)
    logits = logits + b_ref[...]
    w = pos_ref[0]
    bat = bat_ref[0]
    gids = lax.broadcasted_iota(jnp.int32, (G, 1), 0)
    mask = jnp.where(bat == gids, 1.0, 0.0) * w
    sums[...] += jnp.dot(mask, logits, preferred_element_type=jnp.float32)
    cnts[...] += jnp.broadcast_to(
        jnp.sum(mask, axis=1, keepdims=True), cnts.shape)

    @pl.when(i == nb - 1)
    def _fin():
        pooled = sums[...] / jnp.maximum(cnts[...], 1.0)
        pooled_ref[...] = pooled
        gp = _gelu(pooled)
        score_ref[...] = (
            jnp.dot(gp, fw_ref[...], preferred_element_type=jnp.float32)
            + fb_ref[...])


def _pool(parts, dens, b, batch3, pos3, fc_w, fc_b, N, BN):
    DH = b.shape[0]
    nb = N // BN
    return pl.pallas_call(
        functools.partial(_pool_body, nb),
        grid=(nb,),
        in_specs=[
            pl.BlockSpec((2, BN, DH), lambda i: (0, i, 0)),
            pl.BlockSpec((2, BN, 1), lambda i: (0, i, 0)),
            pl.BlockSpec((1, DH), lambda i: (0, 0)),
            pl.BlockSpec((1, 1, BN), lambda i: (i, 0, 0)),
            pl.BlockSpec((1, 1, BN), lambda i: (i, 0, 0)),
            pl.BlockSpec((DH, 1), lambda i: (0, 0)),
            pl.BlockSpec((1, 1), lambda i: (0, 0)),
        ],
        out_specs=[
            pl.BlockSpec((G, DH), lambda i: (0, 0)),
            pl.BlockSpec((G, 1), lambda i: (0, 0)),
        ],
        out_shape=[
            jax.ShapeDtypeStruct((G, DH), jnp.float32),
            jax.ShapeDtypeStruct((G, 1), jnp.float32),
        ],
        scratch_shapes=[
            pltpu.VMEM((G, DH), jnp.float32),
            pltpu.VMEM((G, DH), jnp.float32),
        ],
    )(parts, dens, b.reshape(1, DH), batch3, pos3, fc_w,
      fc_b.reshape(1, 1))


def _make_sc_edge(N, NPD, DH, NB, E_tot):
    TR = NPD // NSUB
    mesh = plsc.VectorSubcoreMesh(core_axis_name="c", subcore_axis_name="s")

    @functools.partial(
        pl.kernel,
        out_type=(
            jax.ShapeDtypeStruct((NCORES * NPD, DH), jnp.float32),
            jax.ShapeDtypeStruct((NCORES * NPD,), jnp.float32),
        ),
        mesh=mesh,
        compiler_params=pltpu.CompilerParams(needs_layout_passes=False),
        scratch_types=[
            pltpu.VMEM((NB, K), jnp.int32),
            pltpu.VMEM((NB, K), jnp.int32),
            pltpu.VMEM((2, K), jnp.float32),
            pltpu.VMEM((2, K), jnp.float32),
            pltpu.VMEM((2, K), jnp.float32),
            pltpu.VMEM((2, K, DH), jnp.float32),
            pltpu.VMEM((128,), jnp.float32),
            pltpu.VMEM((128,), jnp.float32),
            pltpu.VMEM((TR,), jnp.float32),
            pltpu.VMEM_SHARED((NPD, DH), jnp.float32),
            pltpu.VMEM_SHARED((NPD,), jnp.float32),
            [pltpu.SemaphoreType.DMA] * 2,
            [pltpu.SemaphoreType.DMA] * 2,
            [pltpu.SemaphoreType.DMA] * 2,
            [pltpu.SemaphoreType.DMA] * 2,
            [pltpu.SemaphoreType.DMA] * 2,
        ],
    )
    def edge_kernel(h_hbm, als_hbm, ald_hbm, mxs_hbm, mxd_hbm,
                    srcr, dstr, zrow, zden,
                    accp, denp,
                    src_v, dst_v, asg, adg, exb, rows_v, shs, shd,
                    den_stage, acc_sh, den_sh,
                    sem_as, sem_ad, sem_g, sem_dn, sem_sc):
        cid = lax.axis_index("c")
        sid = lax.axis_index("s")
        wid = cid * NSUB + sid

        pltpu.sync_copy(zrow, acc_sh.at[pl.ds(sid * TR, TR)])
        pltpu.sync_copy(zden, den_stage)
        pltpu.sync_copy(den_stage, den_sh.at[pl.ds(sid * TR, TR)])

        pltpu.sync_copy(srcr.at[wid], src_v)
        pltpu.sync_copy(dstr.at[wid], dst_v)
        pltpu.sync_copy(mxs_hbm, shs)
        pltpu.sync_copy(mxd_hbm, shd)
        shift = shs[pl.ds(0, 16)] + shd[pl.ds(0, 16)]
        plsc.subcore_barrier()

        def _issue_gathers(j, b):
            pltpu.async_copy(als_hbm.at[src_v.at[j]], asg.at[b], sem_as[b])
            pltpu.async_copy(ald_hbm.at[dst_v.at[j]], adg.at[b], sem_ad[b])
            pltpu.async_copy(h_hbm.at[src_v.at[j]], rows_v.at[b], sem_g[b])

        _issue_gathers(0, 0)

        def _pair(jo, carry):
            for b in range(2):
                j = 2 * jo + b
                ob = 1 - b

                @pl.when(j >= 1)
                def _w_sc():
                    pltpu.make_async_copy(
                        rows_v.at[ob], acc_sh.at[dst_v.at[j]],
                        sem_sc[ob]).wait()

                @pl.when(j + 1 < NB)
                def _pre():
                    _issue_gathers(j + 1, ob)

                pltpu.make_async_copy(
                    als_hbm.at[src_v.at[j]], asg.at[b], sem_as[b]).wait()
                pltpu.make_async_copy(
                    ald_hbm.at[dst_v.at[j]], adg.at[b], sem_ad[b]).wait()
                pltpu.make_async_copy(
                    h_hbm.at[src_v.at[j]], rows_v.at[b], sem_g[b]).wait()

                @pl.when(j >= 2)
                def _w_dn():
                    pltpu.make_async_copy(
                        exb.at[b], den_sh.at[dst_v.at[j]], sem_dn[b]).wait()

                for g in range(K // 16):
                    sl = pl.ds(g * 16, 16)
                    e = asg[b, sl] + adg[b, sl]
                    e = jnp.where(e >= 0.0, e, 0.2 * e) - shift
                    ex = jnp.exp(e)
                    eid = ((wid * NB + j) * K + g * 16
                           + lax.broadcasted_iota(jnp.int32, (16,), 0))
                    exb[b, sl] = jnp.where(eid < E_tot, ex, 0.0)

                def _scale(g2, c2):
                    exg = exb[b, pl.ds(g2 * 16, 16)]
                    for kk in range(16):
                        sv16 = jnp.full((16,), exg[kk], jnp.float32)
                        row = g2 * 16 + kk
                        for c in range(DH // 16):
                            rows_v[b, row, pl.ds(c * 16, 16)] = (
                                rows_v[b, row, pl.ds(c * 16, 16)] * sv16)
                    return c2

                lax.fori_loop(0, K // 16, _scale, 0)

                pltpu.async_copy(exb.at[b], den_sh.at[dst_v.at[j]],
                                 sem_dn[b], add=True)
                pltpu.async_copy(rows_v.at[b], acc_sh.at[dst_v.at[j]],
                                 sem_sc[b], add=True)
            return carry

        lax.fori_loop(0, NB // 2, _pair, 0)

        pltpu.make_async_copy(rows_v.at[1], acc_sh.at[dst_v.at[0]],
                              sem_sc[1]).wait()
        for b in range(2):
            pltpu.make_async_copy(exb.at[b], den_sh.at[dst_v.at[0]],
                                  sem_dn[b]).wait()
        plsc.subcore_barrier()

        pltpu.sync_copy(acc_sh.at[pl.ds(sid * TR, TR)],
                        accp.at[pl.ds(cid * NPD + sid * TR, TR)])
        pltpu.sync_copy(den_sh.at[pl.ds(sid * TR, TR)], den_stage)
        pltpu.sync_copy(den_stage,
                        denp.at[pl.ds(cid * NPD + sid * TR, TR)])

    return edge_kernel


def kernel(x, edge_index, batch, pos, num_graphs, W1, att_src1, att_dst1, b1,
           W2, att_src2, att_dst2, b2, fc_w, fc_b):
    N, DIN = x.shape
    DH = W1.shape[1]
    E = edge_index.shape[1]
    E_tot = E + N
    BN = 1000

    NB = -(-E_tot // (NW * K))
    EPAD = NW * NB * K
    TRD = ((-(-N // NSUB)) + 7) // 8 * 8
    NPD = NSUB * TRD

    loop = jnp.arange(N, dtype=jnp.int32)
    pad = jnp.zeros((EPAD - E_tot,), dtype=jnp.int32)
    srcr = jnp.concatenate([edge_index[0], loop, pad]).reshape(NW, NB, K)
    dstr = jnp.concatenate([edge_index[1], loop, pad]).reshape(NW, NB, K)
    zrow = jnp.zeros((NPD // NSUB, DH), jnp.float32)
    zden = jnp.zeros((TRD,), jnp.float32)
    batch3 = batch.astype(jnp.int32).reshape(N // BN, 1, BN)
    pos3 = pos.astype(jnp.float32).reshape(N // BN, 1, BN)

    sc_edge = _make_sc_edge(N, NPD, DH, NB, E_tot)

    h1, als1, ald1, mxs1, mxd1 = _mm1(x, W1, att_src1, att_dst1, BN)
    accp1, denp1 = sc_edge(h1, als1.reshape(N), ald1.reshape(N),
                           mxs1.reshape(128), mxd1.reshape(128),
                           srcr, dstr, zrow, zden)
    parts1 = accp1.reshape(NCORES, NPD, DH)[:, :N]
    dens1 = denp1.reshape(NCORES, NPD, 1)[:, :N]

    h2, als2, ald2, mxs2, mxd2 = _mm2(parts1, dens1, b1, W2,
                                      att_src2, att_dst2, N, BN)
    accp2, denp2 = sc_edge(h2, als2.reshape(N), ald2.reshape(N),
                           mxs2.reshape(128), mxd2.reshape(128),
                           srcr, dstr, zrow, zden)
    parts2 = accp2.reshape(NCORES, NPD, DH)[:, :N]
    dens2 = denp2.reshape(NCORES, NPD, 1)[:, :N]

    pooled, score = _pool(parts2, dens2, b2, batch3, pos3, fc_w, fc_b, N, BN)
    return (score.reshape(1, G), pooled)


# speedup vs baseline: 22.6626x; 1.1905x over previous
import functools
import math

import jax
import jax.numpy as jnp
from jax import lax
from jax.experimental import pallas as pl
from jax.experimental.pallas import tpu as pltpu
from jax.experimental.pallas import tpu_sc as plsc

G = 8
NCORES = 2
NSUB = 16
NW = NCORES * NSUB
K = 64


def _gelu(x):
    return x * 0.5 * (1.0 + lax.erf(x / math.sqrt(2.0)))


def _mm1_body(x_ref, w_ref, as_ref, ad_ref,
              h_ref, als_ref, ald_ref, mxs_ref, mxd_ref):
    i = pl.program_id(0)

    @pl.when(i == 0)
    def _init():
        mxs_ref[...] = jnp.full_like(mxs_ref, -1e30)
        mxd_ref[...] = jnp.full_like(mxd_ref, -1e30)

    h = jnp.dot(x_ref[...], w_ref[...], preferred_element_type=jnp.float32)
    h_ref[...] = h
    als = jnp.dot(h, as_ref[...], preferred_element_type=jnp.float32)
    ald = jnp.dot(h, ad_ref[...], preferred_element_type=jnp.float32)
    als_ref[...] = als
    ald_ref[...] = ald
    mxs_ref[...] = jnp.maximum(mxs_ref[...], jnp.max(als))
    mxd_ref[...] = jnp.maximum(mxd_ref[...], jnp.max(ald))


def _mm1(x, W, a_s, a_d, BN):
    N, DIN = x.shape
    DH = W.shape[1]
    return pl.pallas_call(
        _mm1_body,
        grid=(N // BN,),
        in_specs=[
            pl.BlockSpec((BN, DIN), lambda i: (i, 0)),
            pl.BlockSpec((DIN, DH), lambda i: (0, 0)),
            pl.BlockSpec((DH, 1), lambda i: (0, 0)),
            pl.BlockSpec((DH, 1), lambda i: (0, 0)),
        ],
        out_specs=[
            pl.BlockSpec((BN, DH), lambda i: (i, 0)),
            pl.BlockSpec((BN, 1), lambda i: (i, 0)),
            pl.BlockSpec((BN, 1), lambda i: (i, 0)),
            pl.BlockSpec((1, 128), lambda i: (0, 0)),
            pl.BlockSpec((1, 128), lambda i: (0, 0)),
        ],
        out_shape=[
            jax.ShapeDtypeStruct((N, DH), jnp.float32),
            jax.ShapeDtypeStruct((N, 1), jnp.float32),
            jax.ShapeDtypeStruct((N, 1), jnp.float32),
            jax.ShapeDtypeStruct((1, 128), jnp.float32),
            jax.ShapeDtypeStruct((1, 128), jnp.float32),
        ],
    )(x, W, a_s.reshape(DH, 1), a_d.reshape(DH, 1))


def _selfloop(alsp_ref, aldp_ref, mxsp_ref, mxdp_ref):
    e = alsp_ref[...] + aldp_ref[...]
    e = jnp.where(e >= 0.0, e, 0.2 * e)
    shift = mxsp_ref[0, 0] + mxdp_ref[0, 0]
    return jnp.exp(e - shift)


def _mm2_body(p_ref, d_ref, b_ref, w_ref, as_ref, ad_ref,
              h1_ref, alsp_ref, aldp_ref, mxsp_ref, mxdp_ref,
              h2_ref, als_ref, ald_ref, mxs_ref, mxd_ref):
    i = pl.program_id(0)

    @pl.when(i == 0)
    def _init():
        mxs_ref[...] = jnp.full_like(mxs_ref, -1e30)
        mxd_ref[...] = jnp.full_like(mxd_ref, -1e30)

    agg = p_ref[0] + p_ref[1]
    den = d_ref[0] + d_ref[1]
    out1 = agg / jnp.maximum(den, 1e-16) + b_ref[...]
    g = _gelu(out1)
    h2 = jnp.dot(g, w_ref[...], preferred_element_type=jnp.float32)
    h2_ref[...] = h2
    als = jnp.dot(h2, as_ref[...], preferred_element_type=jnp.float32)
    ald = jnp.dot(h2, ad_ref[...], preferred_element_type=jnp.float32)
    als_ref[...] = als
    ald_ref[...] = ald
    mxs_ref[...] = jnp.maximum(mxs_ref[...], jnp.max(als))
    mxd_ref[...] = jnp.maximum(mxd_ref[...], jnp.max(ald))


def _mm2(parts, dens, b, W, a_s, a_d, h1, als1, ald1, mxs1, mxd1, N, BN):
    DH = W.shape[0]
    return pl.pallas_call(
        _mm2_body,
        grid=(N // BN,),
        in_specs=[
            pl.BlockSpec((2, BN, DH), lambda i: (0, i, 0)),
            pl.BlockSpec((2, BN, 1), lambda i: (0, i, 0)),
            pl.BlockSpec((1, DH), lambda i: (0, 0)),
            pl.BlockSpec((DH, DH), lambda i: (0, 0)),
            pl.BlockSpec((DH, 1), lambda i: (0, 0)),
            pl.BlockSpec((DH, 1), lambda i: (0, 0)),
            pl.BlockSpec((BN, DH), lambda i: (i, 0)),
            pl.BlockSpec((BN, 1), lambda i: (i, 0)),
            pl.BlockSpec((BN, 1), lambda i: (i, 0)),
            pl.BlockSpec((1, 128), lambda i: (0, 0)),
            pl.BlockSpec((1, 128), lambda i: (0, 0)),
        ],
        out_specs=[
            pl.BlockSpec((BN, DH), lambda i: (i, 0)),
            pl.BlockSpec((BN, 1), lambda i: (i, 0)),
            pl.BlockSpec((BN, 1), lambda i: (i, 0)),
            pl.BlockSpec((1, 128), lambda i: (0, 0)),
            pl.BlockSpec((1, 128), lambda i: (0, 0)),
        ],
        out_shape=[
            jax.ShapeDtypeStruct((N, DH), jnp.float32),
            jax.ShapeDtypeStruct((N, 1), jnp.float32),
            jax.ShapeDtypeStruct((N, 1), jnp.float32),
            jax.ShapeDtypeStruct((1, 128), jnp.float32),
            jax.ShapeDtypeStruct((1, 128), jnp.float32),
        ],
    )(parts, dens, b.reshape(1, DH), W, a_s.reshape(DH, 1), a_d.reshape(DH, 1),
      h1, als1, ald1, mxs1, mxd1)


def _pool_body(nb, p_ref, d_ref, b_ref, bat_ref, pos_ref, fw_ref, fb_ref,
               h2_ref, alsp_ref, aldp_ref, mxsp_ref, mxdp_ref,
               pooled_ref, score_ref, sums, cnts):
    i = pl.program_id(0)

    @pl.when(i == 0)
    def _init():
        sums[...] = jnp.zeros_like(sums)
        cnts[...] = jnp.zeros_like(cnts)

    logits = (p_ref[0] + p_ref[1]) / jnp.maximum(d_ref[0] + d_ref[1], 1e-16)
    logits = logits + b_ref[...]
    w = pos_ref[0]
    bat = bat_ref[0]
    gids = lax.broadcasted_iota(jnp.int32, (G, 1), 0)
    mask = jnp.where(bat == gids, 1.0, 0.0) * w
    sums[...] += jnp.dot(mask, logits, preferred_element_type=jnp.float32)
    cnts[...] += jnp.broadcast_to(
        jnp.sum(mask, axis=1, keepdims=True), cnts.shape)

    @pl.when(i == nb - 1)
    def _fin():
        pooled = sums[...] / jnp.maximum(cnts[...], 1.0)
        pooled_ref[...] = pooled
        gp = _gelu(pooled)
        score_ref[...] = (
            jnp.dot(gp, fw_ref[...], preferred_element_type=jnp.float32)
            + fb_ref[...])


def _pool(parts, dens, b, batch3, pos3, fc_w, fc_b,
          h2, als2, ald2, mxs2, mxd2, N, BN):
    DH = b.shape[0]
    nb = N // BN
    return pl.pallas_call(
        functools.partial(_pool_body, nb),
        grid=(nb,),
        in_specs=[
            pl.BlockSpec((2, BN, DH), lambda i: (0, i, 0)),
            pl.BlockSpec((2, BN, 1), lambda i: (0, i, 0)),
            pl.BlockSpec((1, DH), lambda i: (0, 0)),
            pl.BlockSpec((1, 1, BN), lambda i: (i, 0, 0)),
            pl.BlockSpec((1, 1, BN), lambda i: (i, 0, 0)),
            pl.BlockSpec((DH, 1), lambda i: (0, 0)),
            pl.BlockSpec((1, 1), lambda i: (0, 0)),
            pl.BlockSpec((BN, DH), lambda i: (i, 0)),
            pl.BlockSpec((BN, 1), lambda i: (i, 0)),
            pl.BlockSpec((BN, 1), lambda i: (i, 0)),
            pl.BlockSpec((1, 128), lambda i: (0, 0)),
            pl.BlockSpec((1, 128), lambda i: (0, 0)),
        ],
        out_specs=[
            pl.BlockSpec((G, DH), lambda i: (0, 0)),
            pl.BlockSpec((G, 1), lambda i: (0, 0)),
        ],
        out_shape=[
            jax.ShapeDtypeStruct((G, DH), jnp.float32),
            jax.ShapeDtypeStruct((G, 1), jnp.float32),
        ],
        scratch_shapes=[
            pltpu.VMEM((G, DH), jnp.float32),
            pltpu.VMEM((G, DH), jnp.float32),
        ],
    )(parts, dens, b.reshape(1, DH), batch3, pos3, fc_w,
      fc_b.reshape(1, 1), h2, als2, ald2, mxs2, mxd2)


def _make_sc_edge(N, NPD, DH, NB):
    TR = NPD // NSUB
    mesh = plsc.VectorSubcoreMesh(core_axis_name="c", subcore_axis_name="s")

    @functools.partial(
        pl.kernel,
        out_type=(
            jax.ShapeDtypeStruct((NCORES * NPD, DH), jnp.float32),
            jax.ShapeDtypeStruct((NCORES * NPD,), jnp.float32),
        ),
        mesh=mesh,
        compiler_params=pltpu.CompilerParams(needs_layout_passes=False),
        scratch_types=[
            pltpu.VMEM((NB // 2, 2 * K), jnp.int32),
            pltpu.VMEM((NB // 2, 2 * K), jnp.int32),
            pltpu.VMEM((4, K), jnp.float32),
            pltpu.VMEM((4, K), jnp.float32),
            pltpu.VMEM((4, K), jnp.float32),
            pltpu.VMEM((4, K, DH), jnp.float32),
            pltpu.VMEM((128,), jnp.float32),
            pltpu.VMEM((128,), jnp.float32),
            pltpu.VMEM((TR,), jnp.float32),
            pltpu.VMEM_SHARED((NPD, DH), jnp.float32),
            pltpu.VMEM_SHARED((NPD,), jnp.float32),
            [pltpu.SemaphoreType.DMA] * 4,
            [pltpu.SemaphoreType.DMA] * 4,
            [pltpu.SemaphoreType.DMA] * 4,
            [pltpu.SemaphoreType.DMA] * 4,
            [pltpu.SemaphoreType.DMA] * 4,
        ],
    )
    def edge_kernel(h_hbm, als_hbm, ald_hbm, mxs_hbm, mxd_hbm,
                    srcr, dstr, zrow, zden,
                    accp, denp,
                    src_v, dst_v, asg, adg, exb, rows_v, shs, shd,
                    den_stage, acc_sh, den_sh,
                    sem_as, sem_ad, sem_g, sem_dn, sem_sc):
        cid = lax.axis_index("c")
        sid = lax.axis_index("s")
        wid = cid * NSUB + sid

        pltpu.sync_copy(zrow, acc_sh.at[pl.ds(sid * TR, TR)])
        pltpu.sync_copy(zden, den_stage)
        pltpu.sync_copy(den_stage, den_sh.at[pl.ds(sid * TR, TR)])

        pltpu.sync_copy(srcr.at[wid], src_v)
        pltpu.sync_copy(dstr.at[wid], dst_v)
        pltpu.sync_copy(mxs_hbm, shs)
        pltpu.sync_copy(mxd_hbm, shd)
        shift = shs[pl.ds(0, 16)] + shd[pl.ds(0, 16)]
        plsc.subcore_barrier()

        def _sidx(jo2, b):
            return src_v.at[2 * jo2 + b // 2, pl.ds((b % 2) * K, K)]

        def _didx(jo2, b):
            return dst_v.at[2 * jo2 + b // 2, pl.ds((b % 2) * K, K)]

        def _issue_gathers(jo2, b):
            bb = b % 4
            pltpu.async_copy(als_hbm.at[_sidx(jo2, b)], asg.at[bb],
                             sem_as[bb])
            pltpu.async_copy(ald_hbm.at[_didx(jo2, b)], adg.at[bb],
                             sem_ad[bb])
            pltpu.async_copy(h_hbm.at[_sidx(jo2, b)], rows_v.at[bb],
                             sem_g[bb])

        _issue_gathers(0, 0)
        _issue_gathers(0, 1)

        def _quad(jo, carry):
            for b in range(4):
                j = 4 * jo + b

                pltpu.make_async_copy(
                    als_hbm.at[_sidx(jo, b)], asg.at[b], sem_as[b]).wait()
                pltpu.make_async_copy(
                    ald_hbm.at[_didx(jo, b)], adg.at[b], sem_ad[b]).wait()
                pltpu.make_async_copy(
                    h_hbm.at[_sidx(jo, b)], rows_v.at[b], sem_g[b]).wait()

                nb2 = (b + 2) % 4

                @pl.when(j >= 2)
                def _w_sc():
                    pltpu.make_async_copy(
                        rows_v.at[nb2], acc_sh.at[_didx(jo, b)],
                        sem_sc[nb2]).wait()

                @pl.when(j + 2 < NB)
                def _pre():
                    if b < 2:
                        _issue_gathers(jo, b + 2)
                    else:
                        _issue_gathers(jo + 1, b - 2)

                @pl.when(j >= 4)
                def _w_dn():
                    pltpu.make_async_copy(
                        exb.at[b], den_sh.at[_didx(jo, b)], sem_dn[b]).wait()

                for g in range(K // 16):
                    sl = pl.ds(g * 16, 16)
                    e = asg[b, sl] + adg[b, sl]
                    e = jnp.where(e >= 0.0, e, 0.2 * e) - shift
                    exb[b, sl] = jnp.exp(e)

                pltpu.async_copy(exb.at[b], den_sh.at[_didx(jo, b)],
                                 sem_dn[b], add=True)

                def _scale(g2, c2):
                    exg = exb[b, pl.ds(g2 * 16, 16)]
                    for kk in range(16):
                        sv16 = jnp.full((16,), exg[kk], jnp.float32)
                        row = g2 * 16 + kk
                        for c in range(DH // 16):
                            rows_v[b, row, pl.ds(c * 16, 16)] = (
                                rows_v[b, row, pl.ds(c * 16, 16)] * sv16)
                    return c2

                lax.fori_loop(0, K // 16, _scale, 0)

                pltpu.async_copy(rows_v.at[b], acc_sh.at[_didx(jo, b)],
                                 sem_sc[b], add=True)
            return carry

        lax.fori_loop(0, NB // 4, _quad, 0)

        for b in (2, 3):
            pltpu.make_async_copy(rows_v.at[b], acc_sh.at[_didx(0, b)],
                                  sem_sc[b]).wait()
        for b in range(4):
            pltpu.make_async_copy(exb.at[b], den_sh.at[_didx(0, b)],
                                  sem_dn[b]).wait()
        plsc.subcore_barrier()

        pltpu.sync_copy(acc_sh.at[pl.ds(sid * TR, TR)],
                        accp.at[pl.ds(cid * NPD + sid * TR, TR)])
        pltpu.sync_copy(den_sh.at[pl.ds(sid * TR, TR)], den_stage)
        pltpu.sync_copy(den_stage,
                        denp.at[pl.ds(cid * NPD + sid * TR, TR)])

    return edge_kernel


def kernel(x, edge_index, batch, pos, num_graphs, W1, att_src1, att_dst1, b1,
           W2, att_src2, att_dst2, b2, fc_w, fc_b):
    N, DIN = x.shape
    DH = W1.shape[1]
    E = edge_index.shape[1]
    BN = 1000

    E_tot = E + N
    NB = -(-E_tot // (NW * K))
    EPAD = NW * NB * K
    TRD = ((-(-N // NSUB)) + 7) // 8 * 8
    NPD = NSUB * TRD

    npad_e = EPAD - E_tot
    loop = jnp.arange(N, dtype=jnp.int32)
    pad_src = jnp.zeros((npad_e,), dtype=jnp.int32)
    pad_dst = (N + jnp.arange(npad_e, dtype=jnp.int32) % (NPD - N))
    srcr = (jnp.concatenate([edge_index[0], loop, pad_src])
            .reshape(NB, NW, K).swapaxes(0, 1).reshape(NW, NB // 2, 2 * K))
    dstr = (jnp.concatenate([edge_index[1], loop, pad_dst])
            .reshape(NB, NW, K).swapaxes(0, 1).reshape(NW, NB // 2, 2 * K))
    zrow = jnp.zeros((NPD // NSUB, DH), jnp.float32)
    zden = jnp.zeros((TRD,), jnp.float32)
    batch3 = batch.astype(jnp.int32).reshape(N // BN, 1, BN)
    pos3 = pos.astype(jnp.float32).reshape(N // BN, 1, BN)

    sc_edge = _make_sc_edge(N, NPD, DH, NB)

    h1, als1, ald1, mxs1, mxd1 = _mm1(x, W1, att_src1, att_dst1, BN)
    accp1, denp1 = sc_edge(h1, als1.reshape(N), ald1.reshape(N),
                           mxs1.reshape(128), mxd1.reshape(128),
                           srcr, dstr, zrow, zden)
    parts1 = accp1.reshape(NCORES, NPD, DH)
    dens1 = denp1.reshape(NCORES, NPD, 1)

    h2, als2, ald2, mxs2, mxd2 = _mm2(parts1, dens1, b1, W2,
                                      att_src2, att_dst2,
                                      h1, als1, ald1, mxs1, mxd1, N, BN)
    accp2, denp2 = sc_edge(h2, als2.reshape(N), ald2.reshape(N),
                           mxs2.reshape(128), mxd2.reshape(128),
                           srcr, dstr, zrow, zden)
    parts2 = accp2.reshape(NCORES, NPD, DH)
    dens2 = denp2.reshape(NCORES, NPD, 1)

    pooled, score = _pool(parts2, dens2, b2, batch3, pos3, fc_w, fc_b,
                          h2, als2, ald2, mxs2, mxd2, N, BN)
    return (score.reshape(1, G), pooled)
